# baseline (speedup 1.0000x reference)
import functools

import jax
import jax.numpy as jnp
from jax import lax
from jax.experimental import pallas as pl
from jax.experimental.pallas import tpu as pltpu
from jax.experimental.pallas import tpu_sc as plsc

N_EMBD = 320
TIME_STEPS = 1000
BATCH = 16384

_D_PAD = 384
_NC = 2
_NS = 16
_NW = _NC * _NS
_N_SPLIT = 1
_B_SPLIT = BATCH // _N_SPLIT
_B_PER_W = _B_SPLIT // _NW
_CHUNK = 32
_N_CHUNKS = _B_PER_W // _CHUNK
_NBUF = 10


def _make_gather():
    mesh = plsc.VectorSubcoreMesh(core_axis_name="c", subcore_axis_name="s")

    @functools.partial(
        pl.kernel,
        mesh=mesh,
        out_type=jax.ShapeDtypeStruct((_B_SPLIT, _D_PAD), jnp.float32),
        scratch_types=[
            pltpu.VMEM((_B_PER_W,), jnp.int32),
        ]
        + [pltpu.VMEM((_CHUNK, _D_PAD), jnp.float32) for _ in range(_NBUF)]
        + [pltpu.SemaphoreType.DMA for _ in range(2 * _NBUF)],
    )
    def k(t_hbm, pe_hbm, out_hbm, idx_v, *rest):
        bufs = rest[:_NBUF]
        gsems = rest[_NBUF : 2 * _NBUF]
        ssems = rest[2 * _NBUF :]
        wid = lax.axis_index("s") * _NC + lax.axis_index("c")
        base = wid * _B_PER_W
        pltpu.sync_copy(t_hbm.at[pl.ds(base, _B_PER_W)], idx_v)

        def idx_slice(c):
            return idx_v.at[pl.ds(c * _CHUNK, _CHUNK)]

        g = [None] * _N_CHUNKS
        s = [None] * _N_CHUNKS
        for c in range(min(_NBUF, _N_CHUNKS)):
            g[c] = pltpu.async_copy(pe_hbm.at[idx_slice(c)], bufs[c], gsems[c])
        for c in range(_N_CHUNKS):
            b = c % _NBUF
            g[c].wait()
            s[c] = pltpu.async_copy(
                bufs[b], out_hbm.at[pl.ds(base + c * _CHUNK, _CHUNK)], ssems[b]
            )
            n = c + _NBUF
            if n < _N_CHUNKS:
                s[n - _NBUF].wait()
                g[n] = pltpu.async_copy(
                    pe_hbm.at[idx_slice(n)], bufs[n % _NBUF], gsems[n % _NBUF]
                )
        for c in range(max(0, _N_CHUNKS - _NBUF), _N_CHUNKS):
            s[c].wait()

    return k


_gather = _make_gather()


@jax.jit
def kernel(t, pe):
    pe_pad = jnp.pad(pe, ((0, 0), (0, _D_PAD - N_EMBD)))
    return _gather(t.astype(jnp.int32), pe_pad)[:, :N_EMBD]


# speedup vs baseline: 1.0129x; 1.0129x over previous
import functools

import jax
import jax.numpy as jnp
from jax import lax
from jax.experimental import pallas as pl
from jax.experimental.pallas import tpu as pltpu
from jax.experimental.pallas import tpu_sc as plsc

N_EMBD = 320
TIME_STEPS = 1000
BATCH = 16384

_D_PAD = 384
_NC = 2
_NS = 16
_NW = _NC * _NS
_N_SPLIT = 1
_B_SPLIT = BATCH // _N_SPLIT
_B_PER_W = _B_SPLIT // _NW
_CHUNK = 64
_N_CHUNKS = _B_PER_W // _CHUNK
_NBUF = 5


def _make_gather():
    mesh = plsc.VectorSubcoreMesh(core_axis_name="c", subcore_axis_name="s")

    @functools.partial(
        pl.kernel,
        mesh=mesh,
        out_type=jax.ShapeDtypeStruct((_B_SPLIT, _D_PAD), jnp.float32),
        scratch_types=[
            pltpu.VMEM((_B_PER_W,), jnp.int32),
        ]
        + [pltpu.VMEM((_CHUNK, _D_PAD), jnp.float32) for _ in range(_NBUF)]
        + [pltpu.SemaphoreType.DMA for _ in range(2 * _NBUF)],
    )
    def k(t_hbm, pe_hbm, out_hbm, idx_v, *rest):
        bufs = rest[:_NBUF]
        gsems = rest[_NBUF : 2 * _NBUF]
        ssems = rest[2 * _NBUF :]
        wid = lax.axis_index("s") * _NC + lax.axis_index("c")
        base = wid * _B_PER_W
        pltpu.sync_copy(t_hbm.at[pl.ds(base, _B_PER_W)], idx_v)

        def idx_slice(c):
            return idx_v.at[pl.ds(c * _CHUNK, _CHUNK)]

        g = [None] * _N_CHUNKS
        s = [None] * _N_CHUNKS
        for c in range(min(_NBUF, _N_CHUNKS)):
            g[c] = pltpu.async_copy(pe_hbm.at[idx_slice(c)], bufs[c], gsems[c])
        for c in range(_N_CHUNKS):
            b = c % _NBUF
            g[c].wait()
            s[c] = pltpu.async_copy(
                bufs[b], out_hbm.at[pl.ds(base + c * _CHUNK, _CHUNK)], ssems[b]
            )
            n = c + _NBUF
            if n < _N_CHUNKS:
                s[n - _NBUF].wait()
                g[n] = pltpu.async_copy(
                    pe_hbm.at[idx_slice(n)], bufs[n % _NBUF], gsems[n % _NBUF]
                )
        for c in range(max(0, _N_CHUNKS - _NBUF), _N_CHUNKS):
            s[c].wait()

    return k


_gather = _make_gather()


@jax.jit
def kernel(t, pe):
    pe_pad = jnp.pad(pe, ((0, 0), (0, _D_PAD - N_EMBD)))
    return _gather(t.astype(jnp.int32), pe_pad)[:, :N_EMBD]
